# Initial kernel scaffold; baseline (speedup 1.0000x reference)
import jax
import jax.numpy as jnp
from jax.experimental import pallas as pl


def kernel(x):
    raise NotImplementedError("write your pallas kernel here")


# speedup vs baseline: 43.9889x; 43.9889x over previous
import functools

import jax
import jax.numpy as jnp
from jax import lax
from jax.experimental import pallas as pl
from jax.experimental.pallas import tpu as pltpu
from jax.experimental.pallas import tpu_sc as plsc

NBINS = 256
NPIX = 512 * 512
CHUNK = 32768
NCHUNKS = NPIX // CHUNK
VECS = CHUNK // 16
NCORES = 2
NSUBCORES = 16


def _hist_body(x_hbm, hist_hbm, buf0, buf1, hist, outv, sem0, sem1):
    wid = lax.axis_index("s") * NCORES + lax.axis_index("c")
    lane = lax.iota(jnp.int32, 16)
    ones = jnp.ones((16,), jnp.float32)

    def zero_body(b, carry):
        hist[b, :] = jnp.zeros((16,), jnp.float32)
        return carry

    lax.fori_loop(0, NBINS, zero_body, 0)

    bufs = (buf0, buf1)
    sems = (sem0, sem1)

    def chunk_src(c):
        return x_hbm.at[wid, pl.ds(c * CHUNK, CHUNK)]

    pltpu.async_copy(chunk_src(0), bufs[0], sems[0])

    for c in range(NCHUNKS):
        buf = bufs[c % 2]
        sem = sems[c % 2]
        if c + 1 < NCHUNKS:
            pltpu.async_copy(chunk_src(c + 1), bufs[(c + 1) % 2], sems[(c + 1) % 2])
        pltpu.make_async_copy(chunk_src(c), buf, sem).wait()

        def scat_body(i, carry):
            v = buf[pl.ds(i * 16, 16)]
            idx = v.astype(jnp.int32)
            plsc.addupdate_scatter(hist, [idx, lane], ones)
            return carry

        lax.fori_loop(0, VECS, scat_body, 0)

    def red_body(g, carry):
        acc = jnp.zeros((16,), jnp.float32)
        for r in range(16):
            s = jnp.sum(hist[g * 16 + r, :])
            acc = jnp.where(lane == r, s, acc)
        outv[pl.ds(g * 16, 16)] = acc
        return carry

    lax.fori_loop(0, 16, red_body, 0)

    pltpu.sync_copy(outv, hist_hbm.at[wid])


_hist_kernel = functools.partial(
    pl.kernel,
    out_type=jax.ShapeDtypeStruct((32, NBINS), jnp.float32),
    mesh=plsc.VectorSubcoreMesh(
        core_axis_name="c", subcore_axis_name="s",
        num_cores=NCORES, num_subcores=NSUBCORES,
    ),
    scratch_types=[
        pltpu.VMEM((CHUNK,), jnp.float32),
        pltpu.VMEM((CHUNK,), jnp.float32),
        pltpu.VMEM((NBINS, 16), jnp.float32),
        pltpu.VMEM((NBINS,), jnp.float32),
        pltpu.SemaphoreType.DMA,
        pltpu.SemaphoreType.DMA,
    ],
    compiler_params=pltpu.CompilerParams(needs_layout_passes=False),
)(_hist_body)


def _entropy_body(h_ref, o_ref):
    h = h_ref[...]
    p = h / jnp.sum(h, axis=1, keepdims=True)
    o_ref[...] = -jnp.sum(p * jnp.log2(p + 1e-8), axis=1, keepdims=True)


def kernel(x):
    B = x.shape[0]
    xf = x.reshape(B, NPIX)
    hist = _hist_kernel(xf)
    score = pl.pallas_call(
        _entropy_body,
        out_shape=jax.ShapeDtypeStruct((B, 1), jnp.float32),
    )(hist)
    return score[:, 0]


# speedup vs baseline: 45.5521x; 1.0355x over previous
import functools

import jax
import jax.numpy as jnp
from jax import lax
from jax.experimental import pallas as pl
from jax.experimental.pallas import tpu as pltpu
from jax.experimental.pallas import tpu_sc as plsc

NBINS = 256
NPIX = 512 * 512
CHUNK = 32768
NCHUNKS = NPIX // CHUNK
VECS = CHUNK // 16
UNROLL = 16
NCORES = 2
NSUBCORES = 16


def _hist_body(x_hbm, hist_hbm, buf0, buf1, hist, outv, sem0, sem1):
    wid = lax.axis_index("s") * NCORES + lax.axis_index("c")
    lane = lax.iota(jnp.int32, 16)
    ones = jnp.ones((16,), jnp.float32)

    def zero_body(b, carry):
        hist[b, :] = jnp.zeros((16,), jnp.float32)
        return carry

    lax.fori_loop(0, NBINS, zero_body, 0)

    bufs = (buf0, buf1)
    sems = (sem0, sem1)

    def chunk_src(c):
        return x_hbm.at[wid, pl.ds(c * CHUNK, CHUNK)]

    pltpu.async_copy(chunk_src(0), bufs[0], sems[0])

    for c in range(NCHUNKS):
        buf = bufs[c % 2]
        sem = sems[c % 2]
        if c + 1 < NCHUNKS:
            pltpu.async_copy(chunk_src(c + 1), bufs[(c + 1) % 2], sems[(c + 1) % 2])
        pltpu.make_async_copy(chunk_src(c), buf, sem).wait()

        def scat_body(i, carry):
            base = i * (16 * UNROLL)
            for u in range(UNROLL):
                v = buf[pl.ds(base + u * 16, 16)]
                idx = v.astype(jnp.int32)
                plsc.addupdate_scatter(hist, [idx, lane], ones)
            return carry

        lax.fori_loop(0, VECS // UNROLL, scat_body, 0)

    def red_body(g, carry):
        acc = jnp.zeros((16,), jnp.float32)
        for r in range(16):
            s = jnp.sum(hist[g * 16 + r, :])
            acc = jnp.where(lane == r, s, acc)
        outv[pl.ds(g * 16, 16)] = acc
        return carry

    lax.fori_loop(0, 16, red_body, 0)

    pltpu.sync_copy(outv, hist_hbm.at[wid])


_hist_kernel = functools.partial(
    pl.kernel,
    out_type=jax.ShapeDtypeStruct((32, NBINS), jnp.float32),
    mesh=plsc.VectorSubcoreMesh(
        core_axis_name="c", subcore_axis_name="s",
        num_cores=NCORES, num_subcores=NSUBCORES,
    ),
    scratch_types=[
        pltpu.VMEM((CHUNK,), jnp.float32),
        pltpu.VMEM((CHUNK,), jnp.float32),
        pltpu.VMEM((NBINS, 16), jnp.float32),
        pltpu.VMEM((NBINS,), jnp.float32),
        pltpu.SemaphoreType.DMA,
        pltpu.SemaphoreType.DMA,
    ],
    compiler_params=pltpu.CompilerParams(needs_layout_passes=False),
)(_hist_body)


def _entropy_body(h_ref, o_ref):
    h = h_ref[...]
    p = h / jnp.sum(h, axis=1, keepdims=True)
    o_ref[...] = -jnp.sum(p * jnp.log2(p + 1e-8), axis=1, keepdims=True)


def kernel(x):
    B = x.shape[0]
    xf = x.reshape(B, NPIX)
    hist = _hist_kernel(xf)
    score = pl.pallas_call(
        _entropy_body,
        out_shape=jax.ShapeDtypeStruct((B, 1), jnp.float32),
    )(hist)
    return score[:, 0]
